# baseline (speedup 1.0000x reference)
import functools

import jax
import jax.numpy as jnp
from jax import lax
from jax.experimental import pallas as pl
from jax.experimental.pallas import tpu as pltpu
from jax.experimental.pallas import tpu_sc as plsc

NC = 2
NS = 16
NW = NC * NS
L = 16


def _slabs(total, cap=128):
    out, s0 = [], 0
    while s0 < total:
        n = min(cap, total - s0)
        out.append((s0, n))
        s0 += n
    return out


def _sc_neighbor_sum(nodes_p, adj_flat, x, *, BP, C, chunks0, chunks1, S, D):
    K = S + 1
    pair = (chunks0 + chunks1) * C
    per_w = max(chunks0, chunks1) * C
    mesh = plsc.VectorSubcoreMesh(core_axis_name="c", subcore_axis_name="s")

    @functools.partial(
        pl.kernel,
        out_type=jax.ShapeDtypeStruct((BP, D), jnp.float32),
        mesh=mesh,
        scratch_types=[
            pltpu.VMEM((per_w,), jnp.int32),
            pltpu.VMEM((S * C,), jnp.int32),
            pltpu.VMEM((S * C,), jnp.int32),
            pltpu.VMEM((K * C,), jnp.int32),
            pltpu.VMEM((K * C,), jnp.int32),
            pltpu.VMEM((K * C, D), jnp.float32),
            pltpu.VMEM((K * C, D), jnp.float32),
            pltpu.VMEM((C, D), jnp.float32),
            pltpu.VMEM((C, D), jnp.float32),
            pltpu.SemaphoreType.DMA,
            pltpu.SemaphoreType.DMA,
            pltpu.SemaphoreType.DMA,
            pltpu.SemaphoreType.DMA,
            pltpu.SemaphoreType.DMA,
            pltpu.SemaphoreType.DMA,
        ],
    )
    def sc_kernel(nodes_hbm, adj_hbm, x_hbm, out_hbm, nodes_w,
                  fidx0, fidx1, idxs0, idxs1, rows0, rows1, acc0, acc1,
                  sem_a0, sem_a1, sem_x0, sem_x1, sem_w0, sem_w1):
        fidx = (fidx0, fidx1)
        idxs = (idxs0, idxs1)
        rows = (rows0, rows1)
        acc = (acc0, acc1)
        sem_a = (sem_a0, sem_a1)
        sem_x = (sem_x0, sem_x1)
        sem_w = (sem_w0, sem_w1)

        cid = lax.axis_index("c")
        sid = lax.axis_index("s")

        def run(base_w, n_chunks):
            pltpu.sync_copy(nodes_hbm.at[pl.ds(base_w, n_chunks * C)],
                            nodes_w.at[pl.ds(0, n_chunks * C)])

            def build_chunk(g):
                b = g % 2
                for t in range(C // L):
                    n = nodes_w[pl.ds(g * C + t * L, L)]
                    idxs[b][pl.ds(t * L, L)] = n
                    f = n * S
                    for j in range(S):
                        fidx[b][pl.ds(j * C + t * L, L)] = f + j
                return [
                    pltpu.async_copy(adj_hbm.at[fidx[b].at[pl.ds(s0, n)]],
                                     idxs[b].at[pl.ds(C + s0, n)], sem_a[b])
                    for s0, n in _slabs(S * C)
                ]

            def fire_x(g):
                b = g % 2
                return [
                    pltpu.async_copy(x_hbm.at[idxs[b].at[pl.ds(s0, n)]],
                                     rows[b].at[pl.ds(s0, n)], sem_x[b])
                    for s0, n in _slabs(K * C)
                ]

            def accumulate(g):
                b = g % 2

                def body(c, _):
                    for t in range(D // L):
                        sl = pl.ds(t * L, L)
                        v = rows[b][c, sl]
                        for j in range(S):
                            v = v + rows[b][C + j * C + c, sl]
                        acc[b][c, sl] = v
                    return 0

                lax.fori_loop(0, C, body, 0, unroll=False)

            adj_h = {0: build_chunk(0)}
            for h in adj_h[0]:
                h.wait()
            x_h = {0: fire_x(0)}
            if n_chunks > 1:
                adj_h[1] = build_chunk(1)
            w_h = {}

            for g in range(n_chunks):
                b = g % 2
                if g + 1 < n_chunks:
                    for h in adj_h[g + 1]:
                        h.wait()
                    x_h[g + 1] = fire_x(g + 1)
                for h in x_h[g]:
                    h.wait()
                if g >= 2:
                    w_h[g - 2].wait()
                if g + 2 < n_chunks:
                    adj_h[g + 2] = build_chunk(g + 2)
                accumulate(g)
                base = pl.multiple_of(base_w + g * C, 8)
                w_h[g] = pltpu.async_copy(acc[b], out_hbm.at[pl.ds(base, C)],
                                          sem_w[b])
            for g in range(max(0, n_chunks - 2), n_chunks):
                w_h[g].wait()

        @pl.when(cid == 0)
        def _():
            run(pl.multiple_of(sid * pair, 8), chunks0)

        @pl.when(cid == 1)
        def _():
            run(pl.multiple_of(sid * pair + chunks0 * C, 8), chunks1)

    return sc_kernel(nodes_p, adj_flat, x)


def _tc_linear_relu(sums, W, *, B, D, TB, scale):
    def body(s_ref, w_ref, o_ref):
        o_ref[...] = jnp.maximum(
            lax.dot_general(s_ref[...] * scale, w_ref[...],
                            (((1,), (1,)), ((), ())),
                            preferred_element_type=jnp.float32),
            0.0,
        )

    return pl.pallas_call(
        body,
        grid=(B // TB,),
        in_specs=[
            pl.BlockSpec((TB, D), lambda i: (i, 0)),
            pl.BlockSpec((D, D), lambda i: (0, 0)),
        ],
        out_specs=pl.BlockSpec((TB, D), lambda i: (i, 0)),
        out_shape=jax.ShapeDtypeStruct((B, D), jnp.float32),
    )(sums, W)


def kernel(nodes, adj, x, W):
    B = nodes.shape[0]
    S = adj.shape[1]
    D = x.shape[1]

    C = 32
    step = NW * C
    BP = ((B + step - 1) // step) * step
    pair_chunks = BP // (NS * C)
    chunks0 = max(2, int(pair_chunks * 0.42))
    chunks1 = pair_chunks - chunks0

    nodes_p = jnp.pad(nodes, (0, BP - B))
    sums = _sc_neighbor_sum(nodes_p, adj.reshape(-1), x, BP=BP, C=C,
                            chunks0=chunks0, chunks1=chunks1, S=S, D=D)

    return _tc_linear_relu(sums, W, B=B, D=D, TB=400,
                           scale=1.0 / float(S + 1))


# speedup vs baseline: 1.0106x; 1.0106x over previous
import functools

import jax
import jax.numpy as jnp
from jax import lax
from jax.experimental import pallas as pl
from jax.experimental.pallas import tpu as pltpu
from jax.experimental.pallas import tpu_sc as plsc

NC = 2
NS = 16
NW = NC * NS
L = 16


def _slabs(total, cap=128):
    out, s0 = [], 0
    while s0 < total:
        n = min(cap, total - s0)
        out.append((s0, n))
        s0 += n
    return out


def _sc_neighbor_sum(nodes_p, adj_flat, x, *, BP, C, chunks0, chunks1, S, D):
    K = S + 1
    pair = (chunks0 + chunks1) * C
    per_w = max(chunks0, chunks1) * C
    mesh = plsc.VectorSubcoreMesh(core_axis_name="c", subcore_axis_name="s")

    @functools.partial(
        pl.kernel,
        out_type=jax.ShapeDtypeStruct((BP, D), jnp.float32),
        mesh=mesh,
        scratch_types=[
            pltpu.VMEM((per_w,), jnp.int32),
            pltpu.VMEM((S * C,), jnp.int32),
            pltpu.VMEM((S * C,), jnp.int32),
            pltpu.VMEM((K * C,), jnp.int32),
            pltpu.VMEM((K * C,), jnp.int32),
            pltpu.VMEM((K * C, D), jnp.float32),
            pltpu.VMEM((K * C, D), jnp.float32),
            pltpu.VMEM((C, D), jnp.float32),
            pltpu.VMEM((C, D), jnp.float32),
            pltpu.SemaphoreType.DMA,
            pltpu.SemaphoreType.DMA,
            pltpu.SemaphoreType.DMA,
            pltpu.SemaphoreType.DMA,
            pltpu.SemaphoreType.DMA,
            pltpu.SemaphoreType.DMA,
        ],
    )
    def sc_kernel(nodes_hbm, adj_hbm, x_hbm, out_hbm, nodes_w,
                  fidx0, fidx1, idxs0, idxs1, rows0, rows1, acc0, acc1,
                  sem_a0, sem_a1, sem_x0, sem_x1, sem_w0, sem_w1):
        fidx = (fidx0, fidx1)
        idxs = (idxs0, idxs1)
        rows = (rows0, rows1)
        acc = (acc0, acc1)
        sem_a = (sem_a0, sem_a1)
        sem_x = (sem_x0, sem_x1)
        sem_w = (sem_w0, sem_w1)

        cid = lax.axis_index("c")
        sid = lax.axis_index("s")

        def run(base_w, n_chunks):
            pltpu.sync_copy(nodes_hbm.at[pl.ds(base_w, n_chunks * C)],
                            nodes_w.at[pl.ds(0, n_chunks * C)])

            def build_chunk(g):
                b = g % 2
                for t in range(C // L):
                    n = nodes_w[pl.ds(g * C + t * L, L)]
                    idxs[b][pl.ds(t * L, L)] = n
                    f = n * S
                    for j in range(S):
                        fidx[b][pl.ds(j * C + t * L, L)] = f + j
                return [
                    pltpu.async_copy(adj_hbm.at[fidx[b].at[pl.ds(s0, n)]],
                                     idxs[b].at[pl.ds(C + s0, n)], sem_a[b])
                    for s0, n in _slabs(S * C)
                ]

            def fire_x(g):
                b = g % 2
                return [
                    pltpu.async_copy(x_hbm.at[idxs[b].at[pl.ds(s0, n)]],
                                     rows[b].at[pl.ds(s0, n)], sem_x[b])
                    for s0, n in _slabs(K * C)
                ]

            def accumulate(g):
                b = g % 2

                def body(c, _):
                    for t in range(D // L):
                        sl = pl.ds(t * L, L)
                        v = rows[b][c, sl]
                        for j in range(S):
                            v = v + rows[b][C + j * C + c, sl]
                        acc[b][c, sl] = v
                    return 0

                lax.fori_loop(0, C, body, 0, unroll=False)

            adj_h = {0: build_chunk(0)}
            for h in adj_h[0]:
                h.wait()
            x_h = {0: fire_x(0)}
            if n_chunks > 1:
                adj_h[1] = build_chunk(1)
            w_h = {}

            for g in range(n_chunks):
                b = g % 2
                if g + 1 < n_chunks:
                    for h in adj_h[g + 1]:
                        h.wait()
                    x_h[g + 1] = fire_x(g + 1)
                for h in x_h[g]:
                    h.wait()
                if g >= 2:
                    w_h[g - 2].wait()
                if g + 2 < n_chunks:
                    adj_h[g + 2] = build_chunk(g + 2)
                accumulate(g)
                base = pl.multiple_of(base_w + g * C, 8)
                w_h[g] = pltpu.async_copy(acc[b], out_hbm.at[pl.ds(base, C)],
                                          sem_w[b])
            for g in range(max(0, n_chunks - 2), n_chunks):
                w_h[g].wait()

        @pl.when(cid == 0)
        def _():
            run(pl.multiple_of(sid * pair, 8), chunks0)

        @pl.when(cid == 1)
        def _():
            run(pl.multiple_of(sid * pair + chunks0 * C, 8), chunks1)

    return sc_kernel(nodes_p, adj_flat, x)


def _tc_linear_relu(sums, W, *, B, D, TB, scale):
    def body(s_ref, w_ref, o_ref):
        o_ref[...] = jnp.maximum(
            lax.dot_general(s_ref[...] * scale, w_ref[...],
                            (((1,), (1,)), ((), ())),
                            preferred_element_type=jnp.float32),
            0.0,
        )

    return pl.pallas_call(
        body,
        grid=(B // TB,),
        in_specs=[
            pl.BlockSpec((TB, D), lambda i: (i, 0)),
            pl.BlockSpec((D, D), lambda i: (0, 0)),
        ],
        out_specs=pl.BlockSpec((TB, D), lambda i: (i, 0)),
        out_shape=jax.ShapeDtypeStruct((B, D), jnp.float32),
    )(sums, W)


def kernel(nodes, adj, x, W):
    B = nodes.shape[0]
    S = adj.shape[1]
    D = x.shape[1]

    C = 32
    step = NW * C
    BP = ((B + step - 1) // step) * step
    pair_chunks = BP // (NS * C)
    chunks0 = pair_chunks // 2
    chunks1 = pair_chunks - chunks0

    nodes_p = jnp.pad(nodes, (0, BP - B))
    sums = _sc_neighbor_sum(nodes_p, adj.reshape(-1), x, BP=BP, C=C,
                            chunks0=chunks0, chunks1=chunks1, S=S, D=D)

    return _tc_linear_relu(sums, W, B=B, D=D, TB=400,
                           scale=1.0 / float(S + 1))


# speedup vs baseline: 1.0395x; 1.0286x over previous
import functools

import jax
import jax.numpy as jnp
from jax import lax
from jax.experimental import pallas as pl
from jax.experimental.pallas import tpu as pltpu
from jax.experimental.pallas import tpu_sc as plsc

NC = 2
NS = 16
NW = NC * NS
L = 16


def _slabs(total, cap=128):
    out, s0 = [], 0
    while s0 < total:
        n = min(cap, total - s0)
        out.append((s0, n))
        s0 += n
    return out


def _sc_neighbor_sum(nodes_p, adj_flat, x, *, BP, C, n_chunks, S, D):
    K = S + 1
    per_w = BP // NW
    mesh = plsc.VectorSubcoreMesh(core_axis_name="c", subcore_axis_name="s")

    @functools.partial(
        pl.kernel,
        out_type=jax.ShapeDtypeStruct((BP, D), jnp.float32),
        mesh=mesh,
        scratch_types=[
            pltpu.VMEM((per_w,), jnp.int32),
            pltpu.VMEM((S * C,), jnp.int32),
            pltpu.VMEM((S * C,), jnp.int32),
            pltpu.VMEM((K * C,), jnp.int32),
            pltpu.VMEM((K * C,), jnp.int32),
            pltpu.VMEM((K * C, D), jnp.float32),
            pltpu.VMEM((K * C, D), jnp.float32),
            pltpu.VMEM((C, D), jnp.float32),
            pltpu.VMEM((C, D), jnp.float32),
            pltpu.SemaphoreType.DMA,
            pltpu.SemaphoreType.DMA,
            pltpu.SemaphoreType.DMA,
            pltpu.SemaphoreType.DMA,
            pltpu.SemaphoreType.DMA,
            pltpu.SemaphoreType.DMA,
        ],
    )
    def sc_kernel(nodes_hbm, adj_hbm, x_hbm, out_hbm, nodes_w,
                  fidx0, fidx1, idxs0, idxs1, rows0, rows1, acc0, acc1,
                  sem_a0, sem_a1, sem_x0, sem_x1, sem_w0, sem_w1):
        fidx = (fidx0, fidx1)
        idxs = (idxs0, idxs1)
        rows = (rows0, rows1)
        acc = (acc0, acc1)
        sem_a = (sem_a0, sem_a1)
        sem_x = (sem_x0, sem_x1)
        sem_w = (sem_w0, sem_w1)

        wid = lax.axis_index("s") * NC + lax.axis_index("c")
        base_w = pl.multiple_of(wid * per_w, 8)
        pltpu.sync_copy(nodes_hbm.at[pl.ds(base_w, per_w)], nodes_w)

        def build_chunk(g):
            b = g % 2
            for t in range(C // L):
                n = nodes_w[pl.ds(g * C + t * L, L)]
                idxs[b][pl.ds(t * L, L)] = n
                f = n * S
                for j in range(S):
                    fidx[b][pl.ds(j * C + t * L, L)] = f + j
            return [
                pltpu.async_copy(adj_hbm.at[fidx[b].at[pl.ds(s0, n)]],
                                 idxs[b].at[pl.ds(C + s0, n)], sem_a[b])
                for s0, n in _slabs(S * C)
            ]

        def fire_x(g):
            b = g % 2
            return [
                pltpu.async_copy(x_hbm.at[idxs[b].at[pl.ds(s0, n)]],
                                 rows[b].at[pl.ds(s0, n)], sem_x[b])
                for s0, n in _slabs(K * C)
            ]

        def accumulate(g):
            b = g % 2

            def body(c, _):
                for t in range(D // L):
                    sl = pl.ds(t * L, L)
                    v = rows[b][c, sl]
                    for j in range(S):
                        v = v + rows[b][C + j * C + c, sl]
                    acc[b][c, sl] = v
                return 0

            lax.fori_loop(0, C, body, 0, unroll=False)

        adj_h = {0: build_chunk(0)}
        for h in adj_h[0]:
            h.wait()
        x_h = {0: fire_x(0)}
        if n_chunks > 1:
            adj_h[1] = build_chunk(1)
        w_h = {}

        for g in range(n_chunks):
            b = g % 2
            if g + 1 < n_chunks:
                for h in adj_h[g + 1]:
                    h.wait()
                x_h[g + 1] = fire_x(g + 1)
            for h in x_h[g]:
                h.wait()
            if g >= 2:
                w_h[g - 2].wait()
            if g + 2 < n_chunks:
                adj_h[g + 2] = build_chunk(g + 2)
            accumulate(g)
            base = pl.multiple_of(base_w + g * C, 8)
            w_h[g] = pltpu.async_copy(acc[b], out_hbm.at[pl.ds(base, C)],
                                      sem_w[b])
        for g in range(max(0, n_chunks - 2), n_chunks):
            w_h[g].wait()

    return sc_kernel(nodes_p, adj_flat, x)


def _tc_linear_relu(sums, W, *, B, D, TB, scale):
    def body(s_ref, w_ref, o_ref):
        o_ref[...] = jnp.maximum(
            lax.dot_general(s_ref[...] * scale, w_ref[...],
                            (((1,), (1,)), ((), ())),
                            preferred_element_type=jnp.float32),
            0.0,
        )

    return pl.pallas_call(
        body,
        grid=(B // TB,),
        in_specs=[
            pl.BlockSpec((TB, D), lambda i: (i, 0)),
            pl.BlockSpec((D, D), lambda i: (0, 0)),
        ],
        out_specs=pl.BlockSpec((TB, D), lambda i: (i, 0)),
        out_shape=jax.ShapeDtypeStruct((B, D), jnp.float32),
    )(sums, W)


def kernel(nodes, adj, x, W):
    B = nodes.shape[0]
    S = adj.shape[1]
    D = x.shape[1]

    C = 32
    step = NW * C
    BP = ((B + step - 1) // step) * step
    n_chunks = BP // step

    nodes_p = jnp.pad(nodes, (0, BP - B))
    sums = _sc_neighbor_sum(nodes_p, adj.reshape(-1), x, BP=BP, C=C,
                            n_chunks=n_chunks, S=S, D=D)

    return _tc_linear_relu(sums, W, B=B, D=D, TB=400,
                           scale=1.0 / float(S + 1))


# speedup vs baseline: 1.1089x; 1.0667x over previous
import functools

import jax
import jax.numpy as jnp
from jax import lax
from jax.experimental import pallas as pl
from jax.experimental.pallas import tpu as pltpu
from jax.experimental.pallas import tpu_sc as plsc

NC = 2
NS = 16
NW = NC * NS
L = 16


def _slabs(total, cap=128):
    out, s0 = [], 0
    while s0 < total:
        n = min(cap, total - s0)
        out.append((s0, n))
        s0 += n
    return out


def _sc_neighbor_sum(nodes_p, adj_flat, x, *, BP, C, n_chunks, S, D):
    K = S + 1
    per_w = BP // NW
    mesh = plsc.VectorSubcoreMesh(core_axis_name="c", subcore_axis_name="s")

    @functools.partial(
        pl.kernel,
        out_type=jax.ShapeDtypeStruct((BP, D), jnp.float32),
        mesh=mesh,
        scratch_types=[
            pltpu.VMEM((per_w,), jnp.int32),
            pltpu.VMEM((S * C,), jnp.int32),
            pltpu.VMEM((S * C,), jnp.int32),
            pltpu.VMEM((K * C,), jnp.int32),
            pltpu.VMEM((K * C,), jnp.int32),
            pltpu.VMEM((K * C, D), jnp.float32),
            pltpu.VMEM((K * C, D), jnp.float32),
            pltpu.VMEM((C, D), jnp.float32),
            pltpu.VMEM((C, D), jnp.float32),
            pltpu.SemaphoreType.DMA,
            pltpu.SemaphoreType.DMA,
            pltpu.SemaphoreType.DMA,
            pltpu.SemaphoreType.DMA,
            pltpu.SemaphoreType.DMA,
            pltpu.SemaphoreType.DMA,
        ],
    )
    def sc_kernel(nodes_hbm, adj_hbm, x_hbm, out_hbm, nodes_w,
                  fidx0, fidx1, idxs0, idxs1, rows0, rows1, acc0, acc1,
                  sem_a0, sem_a1, sem_x0, sem_x1, sem_w0, sem_w1):
        fidx = (fidx0, fidx1)
        idxs = (idxs0, idxs1)
        rows = (rows0, rows1)
        acc = (acc0, acc1)
        sem_a = (sem_a0, sem_a1)
        sem_x = (sem_x0, sem_x1)
        sem_w = (sem_w0, sem_w1)

        wid = lax.axis_index("s") * NC + lax.axis_index("c")
        base_w = pl.multiple_of(wid * per_w, 8)
        pltpu.sync_copy(nodes_hbm.at[pl.ds(base_w, per_w)], nodes_w)

        def build_chunk(g):
            b = g % 2
            for t in range(C // L):
                n = nodes_w[pl.ds(g * C + t * L, L)]
                idxs[b][pl.ds(t * L, L)] = n
                f = n * S
                for j in range(S):
                    fidx[b][pl.ds(j * C + t * L, L)] = f + j
            return [
                pltpu.async_copy(adj_hbm.at[fidx[b].at[pl.ds(s0, n)]],
                                 idxs[b].at[pl.ds(C + s0, n)], sem_a[b])
                for s0, n in _slabs(S * C)
            ]

        def fire_x(g):
            b = g % 2
            return [
                pltpu.async_copy(x_hbm.at[idxs[b].at[pl.ds(s0, n)]],
                                 rows[b].at[pl.ds(s0, n)], sem_x[b])
                for s0, n in _slabs(K * C)
            ]

        def accumulate(g):
            b = g % 2

            def body(c, _):
                for t in range(D // L):
                    sl = pl.ds(t * L, L)
                    v = rows[b][c, sl]
                    for j in range(S):
                        v = v + rows[b][C + j * C + c, sl]
                    acc[b][c, sl] = v
                return 0

            lax.fori_loop(0, C, body, 0, unroll=False)

        adj_h = {0: build_chunk(0)}
        for h in adj_h[0]:
            h.wait()
        x_h = {0: fire_x(0)}
        if n_chunks > 1:
            adj_h[1] = build_chunk(1)
        w_h = {}

        for g in range(n_chunks):
            b = g % 2
            if g + 1 < n_chunks:
                for h in adj_h[g + 1]:
                    h.wait()
                x_h[g + 1] = fire_x(g + 1)
            for h in x_h[g]:
                h.wait()
            if g >= 2:
                w_h[g - 2].wait()
            if g + 2 < n_chunks:
                adj_h[g + 2] = build_chunk(g + 2)
            accumulate(g)
            base = pl.multiple_of(base_w + g * C, 8)
            w_h[g] = pltpu.async_copy(acc[b], out_hbm.at[pl.ds(base, C)],
                                      sem_w[b])
        for g in range(max(0, n_chunks - 2), n_chunks):
            w_h[g].wait()

    return sc_kernel(nodes_p, adj_flat, x)


def _tc_linear_relu(sums, W, *, B, D, TB, scale):
    def body(s_ref, w_ref, o_ref):
        o_ref[...] = jnp.maximum(
            lax.dot_general(s_ref[...] * scale, w_ref[...],
                            (((1,), (1,)), ((), ())),
                            preferred_element_type=jnp.float32),
            0.0,
        )

    return pl.pallas_call(
        body,
        grid=(B // TB,),
        in_specs=[
            pl.BlockSpec((TB, D), lambda i: (i, 0)),
            pl.BlockSpec((D, D), lambda i: (0, 0)),
        ],
        out_specs=pl.BlockSpec((TB, D), lambda i: (i, 0)),
        out_shape=jax.ShapeDtypeStruct((B, D), jnp.float32),
    )(sums, W)


def kernel(nodes, adj, x, W):
    B = nodes.shape[0]
    S = adj.shape[1]
    D = x.shape[1]

    C = 32
    step = NW * C
    BP = ((B + step - 1) // step) * step
    n_chunks = BP // step

    nodes_p = jnp.pad(nodes, (0, BP - B))
    sums = _sc_neighbor_sum(nodes_p, adj.reshape(-1), x, BP=BP, C=C,
                            n_chunks=n_chunks, S=S, D=D)

    return _tc_linear_relu(sums, W, B=B, D=D, TB=2000,
                           scale=1.0 / float(S + 1))


# speedup vs baseline: 1.1195x; 1.0096x over previous
import functools

import jax
import jax.numpy as jnp
from jax import lax
from jax.experimental import pallas as pl
from jax.experimental.pallas import tpu as pltpu
from jax.experimental.pallas import tpu_sc as plsc

NC = 2
NS = 16
NW = NC * NS
L = 16


def _slabs(total, cap=128):
    out, s0 = [], 0
    while s0 < total:
        n = min(cap, total - s0)
        out.append((s0, n))
        s0 += n
    return out


def _sc_neighbor_sum(nodes_p, adj_flat, x, *, BP, C, n_chunks, S, D):
    K = S + 1
    per_w = BP // NW
    mesh = plsc.VectorSubcoreMesh(core_axis_name="c", subcore_axis_name="s")

    @functools.partial(
        pl.kernel,
        out_type=jax.ShapeDtypeStruct((BP, D), jnp.float32),
        mesh=mesh,
        scratch_types=[
            pltpu.VMEM((per_w,), jnp.int32),
            pltpu.VMEM((S * C,), jnp.int32),
            pltpu.VMEM((S * C,), jnp.int32),
            pltpu.VMEM((K * C,), jnp.int32),
            pltpu.VMEM((K * C,), jnp.int32),
            pltpu.VMEM((K * C, D), jnp.float32),
            pltpu.VMEM((K * C, D), jnp.float32),
            pltpu.VMEM((C, D), jnp.float32),
            pltpu.VMEM((C, D), jnp.float32),
            pltpu.SemaphoreType.DMA,
            pltpu.SemaphoreType.DMA,
            pltpu.SemaphoreType.DMA,
            pltpu.SemaphoreType.DMA,
            pltpu.SemaphoreType.DMA,
            pltpu.SemaphoreType.DMA,
        ],
    )
    def sc_kernel(nodes_hbm, adj_hbm, x_hbm, out_hbm, nodes_w,
                  fidx0, fidx1, idxs0, idxs1, rows0, rows1, acc0, acc1,
                  sem_a0, sem_a1, sem_x0, sem_x1, sem_w0, sem_w1):
        fidx = (fidx0, fidx1)
        idxs = (idxs0, idxs1)
        rows = (rows0, rows1)
        acc = (acc0, acc1)
        sem_a = (sem_a0, sem_a1)
        sem_x = (sem_x0, sem_x1)
        sem_w = (sem_w0, sem_w1)

        wid = lax.axis_index("s") * NC + lax.axis_index("c")
        base_w = pl.multiple_of(wid * per_w, 8)
        pltpu.sync_copy(nodes_hbm.at[pl.ds(base_w, per_w)], nodes_w)

        def build_chunk(g):
            b = g % 2
            for t in range(C // L):
                n = nodes_w[pl.ds(g * C + t * L, L)]
                idxs[b][pl.ds(t * L, L)] = n
                f = n * S
                for j in range(S):
                    fidx[b][pl.ds(j * C + t * L, L)] = f + j
            return [
                pltpu.async_copy(adj_hbm.at[fidx[b].at[pl.ds(s0, n)]],
                                 idxs[b].at[pl.ds(C + s0, n)], sem_a[b])
                for s0, n in _slabs(S * C)
            ]

        def fire_x(g):
            b = g % 2
            return [
                pltpu.async_copy(x_hbm.at[idxs[b].at[pl.ds(s0, n)]],
                                 rows[b].at[pl.ds(s0, n)], sem_x[b])
                for s0, n in _slabs(K * C)
            ]

        def accumulate(g):
            b = g % 2

            def body(c, _):
                for t in range(D // L):
                    sl = pl.ds(t * L, L)
                    v = rows[b][c, sl]
                    for j in range(S):
                        v = v + rows[b][C + j * C + c, sl]
                    acc[b][c, sl] = v
                return 0

            lax.fori_loop(0, C, body, 0, unroll=False)

        adj_h = {0: build_chunk(0)}
        for h in adj_h[0]:
            h.wait()
        x_h = {0: fire_x(0)}
        if n_chunks > 1:
            adj_h[1] = build_chunk(1)
        w_h = {}

        for g in range(n_chunks):
            b = g % 2
            if g + 1 < n_chunks:
                for h in adj_h[g + 1]:
                    h.wait()
                x_h[g + 1] = fire_x(g + 1)
            for h in x_h[g]:
                h.wait()
            if g >= 2:
                w_h[g - 2].wait()
            if g + 2 < n_chunks:
                adj_h[g + 2] = build_chunk(g + 2)
            accumulate(g)
            base = pl.multiple_of(base_w + g * C, 8)
            w_h[g] = pltpu.async_copy(acc[b], out_hbm.at[pl.ds(base, C)],
                                      sem_w[b])
        for g in range(max(0, n_chunks - 2), n_chunks):
            w_h[g].wait()

    return sc_kernel(nodes_p, adj_flat, x)


def _tc_linear_relu(sums, W, *, B, D, TB, scale):
    def body(s_ref, w_ref, o_ref):
        o_ref[...] = jnp.maximum(
            lax.dot_general(s_ref[...] * scale, w_ref[...],
                            (((1,), (1,)), ((), ())),
                            preferred_element_type=jnp.float32),
            0.0,
        )

    return pl.pallas_call(
        body,
        grid=(B // TB,),
        in_specs=[
            pl.BlockSpec((TB, D), lambda i: (i, 0)),
            pl.BlockSpec((D, D), lambda i: (0, 0)),
        ],
        out_specs=pl.BlockSpec((TB, D), lambda i: (i, 0)),
        out_shape=jax.ShapeDtypeStruct((B, D), jnp.float32),
    )(sums, W)


def kernel(nodes, adj, x, W):
    B = nodes.shape[0]
    S = adj.shape[1]
    D = x.shape[1]

    C = 32
    step = NW * C
    BP = ((B + step - 1) // step) * step
    n_chunks = BP // step

    nodes_p = jnp.pad(nodes, (0, BP - B))
    sums = _sc_neighbor_sum(nodes_p, adj.reshape(-1), x, BP=BP, C=C,
                            n_chunks=n_chunks, S=S, D=D)

    return _tc_linear_relu(sums, W, B=B, D=D, TB=5000,
                           scale=1.0 / float(S + 1))


# speedup vs baseline: 1.1239x; 1.0039x over previous
import functools

import jax
import jax.numpy as jnp
from jax import lax
from jax.experimental import pallas as pl
from jax.experimental.pallas import tpu as pltpu
from jax.experimental.pallas import tpu_sc as plsc

NC = 2
NS = 16
NW = NC * NS
L = 16


def _slabs(total, cap=128):
    out, s0 = [], 0
    while s0 < total:
        n = min(cap, total - s0)
        out.append((s0, n))
        s0 += n
    return out


def _sc_neighbor_sum(nodes_p, adj_flat, x, *, BP, C, n_chunks, S, D):
    K = S + 1
    per_w = BP // NW
    mesh = plsc.VectorSubcoreMesh(core_axis_name="c", subcore_axis_name="s")

    @functools.partial(
        pl.kernel,
        out_type=jax.ShapeDtypeStruct((BP, D), jnp.float32),
        mesh=mesh,
        scratch_types=[
            pltpu.VMEM((per_w,), jnp.int32),
            pltpu.VMEM((S * C,), jnp.int32),
            pltpu.VMEM((S * C,), jnp.int32),
            pltpu.VMEM((K * C,), jnp.int32),
            pltpu.VMEM((K * C,), jnp.int32),
            pltpu.VMEM((K * C, D), jnp.float32),
            pltpu.VMEM((K * C, D), jnp.float32),
            pltpu.VMEM((C, D), jnp.float32),
            pltpu.VMEM((C, D), jnp.float32),
            pltpu.SemaphoreType.DMA,
            pltpu.SemaphoreType.DMA,
            pltpu.SemaphoreType.DMA,
            pltpu.SemaphoreType.DMA,
            pltpu.SemaphoreType.DMA,
            pltpu.SemaphoreType.DMA,
        ],
    )
    def sc_kernel(nodes_hbm, adj_hbm, x_hbm, out_hbm, nodes_w,
                  fidx0, fidx1, idxs0, idxs1, rows0, rows1, acc0, acc1,
                  sem_a0, sem_a1, sem_x0, sem_x1, sem_w0, sem_w1):
        fidx = (fidx0, fidx1)
        idxs = (idxs0, idxs1)
        rows = (rows0, rows1)
        acc = (acc0, acc1)
        sem_a = (sem_a0, sem_a1)
        sem_x = (sem_x0, sem_x1)
        sem_w = (sem_w0, sem_w1)

        wid = lax.axis_index("s") * NC + lax.axis_index("c")
        base_w = pl.multiple_of(wid * per_w, 8)
        pltpu.sync_copy(nodes_hbm.at[pl.ds(base_w, per_w)], nodes_w)

        def build_chunk(g):
            b = g % 2
            for t in range(C // L):
                n = nodes_w[pl.ds(g * C + t * L, L)]
                idxs[b][pl.ds(t * L, L)] = n
                f = n * S
                for j in range(S):
                    fidx[b][pl.ds(j * C + t * L, L)] = f + j
            return [
                pltpu.async_copy(adj_hbm.at[fidx[b].at[pl.ds(s0, n)]],
                                 idxs[b].at[pl.ds(C + s0, n)], sem_a[b])
                for s0, n in _slabs(S * C)
            ]

        def fire_x(g):
            b = g % 2
            return [
                pltpu.async_copy(x_hbm.at[idxs[b].at[pl.ds(s0, n)]],
                                 rows[b].at[pl.ds(s0, n)], sem_x[b])
                for s0, n in _slabs(K * C)
            ]

        def accumulate(g):
            b = g % 2

            def body(c, _):
                for t in range(D // L):
                    sl = pl.ds(t * L, L)
                    v = rows[b][c, sl]
                    for j in range(S):
                        v = v + rows[b][C + j * C + c, sl]
                    acc[b][c, sl] = v
                return 0

            lax.fori_loop(0, C, body, 0, unroll=False)

        adj_h = {0: build_chunk(0)}
        for h in adj_h[0]:
            h.wait()
        x_h = {0: fire_x(0)}
        if n_chunks > 1:
            adj_h[1] = build_chunk(1)
        w_h = {}

        for g in range(n_chunks):
            b = g % 2
            if g + 1 < n_chunks:
                for h in adj_h[g + 1]:
                    h.wait()
                x_h[g + 1] = fire_x(g + 1)
            for h in x_h[g]:
                h.wait()
            if g >= 2:
                w_h[g - 2].wait()
            if g + 2 < n_chunks:
                adj_h[g + 2] = build_chunk(g + 2)
            accumulate(g)
            base = pl.multiple_of(base_w + g * C, 8)
            w_h[g] = pltpu.async_copy(acc[b], out_hbm.at[pl.ds(base, C)],
                                      sem_w[b])
        for g in range(max(0, n_chunks - 2), n_chunks):
            w_h[g].wait()

    return sc_kernel(nodes_p, adj_flat, x)


def _tc_linear_relu(sums, W, *, B, D, TB, scale):
    def body(s_ref, w_ref, o_ref):
        o_ref[...] = jnp.maximum(
            lax.dot_general(s_ref[...] * scale, w_ref[...],
                            (((1,), (1,)), ((), ())),
                            preferred_element_type=jnp.float32),
            0.0,
        )

    return pl.pallas_call(
        body,
        grid=(B // TB,),
        in_specs=[
            pl.BlockSpec((TB, D), lambda i: (i, 0)),
            pl.BlockSpec((D, D), lambda i: (0, 0)),
        ],
        out_specs=pl.BlockSpec((TB, D), lambda i: (i, 0)),
        out_shape=jax.ShapeDtypeStruct((B, D), jnp.float32),
    )(sums, W)


def kernel(nodes, adj, x, W):
    B = nodes.shape[0]
    S = adj.shape[1]
    D = x.shape[1]

    C = 32
    step = NW * C
    BP = ((B + step - 1) // step) * step
    n_chunks = BP // step

    nodes_p = jnp.pad(nodes, (0, BP - B))
    sums = _sc_neighbor_sum(nodes_p, adj.reshape(-1), x, BP=BP, C=C,
                            n_chunks=n_chunks, S=S, D=D)

    return _tc_linear_relu(sums, W, B=B, D=D, TB=10000,
                           scale=1.0 / float(S + 1))
